# baseline (speedup 1.0000x reference)
import jax
import jax.numpy as jnp
from jax import lax
from jax.experimental import pallas as pl
from jax.experimental import pallas as _pl_unused
from jax.experimental.pallas import tpu as pltpu

B, S, LANES = 4, 8192, 2048
SB = 64
LB = 2048
NS = S // SB


def _tc_body(x_ref, o_ref, carry_ref):
    s = pl.program_id(1)

    @pl.when(s == 0)
    def _reset():
        carry_ref[...] = jnp.zeros_like(carry_ref)

    x = x_ref[0]
    i = lax.broadcasted_iota(jnp.int32, (SB, SB), 0)
    j = lax.broadcasted_iota(jnp.int32, (SB, SB), 1)
    tri = (i >= j).astype(jnp.float32)
    c = lax.dot(tri, x, preferred_element_type=jnp.float32)
    c = c + carry_ref[0:1, :]
    o_ref[0] = c
    carry_ref[...] = c[SB - 1:SB, :] * jnp.ones((8, 1), jnp.float32)


def _cumsum_tc(x):
    return pl.pallas_call(
        _tc_body,
        grid=(B, NS),
        in_specs=[pl.BlockSpec((1, SB, LB), lambda b, s: (b, s, 0))],
        out_specs=pl.BlockSpec((1, SB, LB), lambda b, s: (b, s, 0)),
        out_shape=jax.ShapeDtypeStruct((B, S, LANES), jnp.float32),
        scratch_shapes=[pltpu.VMEM((8, LB), jnp.float32)],
        compiler_params=pltpu.CompilerParams(
            dimension_semantics=("parallel", "arbitrary"),
        ),
    )(x)


def kernel(masks):
    return _cumsum_tc(masks)


# speedup vs baseline: 1.0173x; 1.0173x over previous
import functools

import jax
import jax.numpy as jnp
from jax import lax
from jax.experimental import pallas as pl
from jax.experimental.pallas import tpu as pltpu
from jax.experimental.pallas import tpu_sc as plsc

B, S, LANES = 4, 8192, 2048
NW = 32
LC = LANES * B // NW
NCHUNK = LC // 16
ST = 64
NTILES = S // ST
NBUF = 2
NGROUPS = NTILES // NBUF

_mesh = plsc.VectorSubcoreMesh(core_axis_name="c", subcore_axis_name="s")


@functools.partial(
    pl.kernel,
    out_type=jax.ShapeDtypeStruct((B, S, LANES), jnp.float32),
    mesh=_mesh,
    scratch_types=[
        pltpu.VMEM((NBUF, ST, LC), jnp.float32),
        pltpu.VMEM((NBUF, ST, LC), jnp.float32),
        [pltpu.SemaphoreType.DMA] * NBUF,
        [pltpu.SemaphoreType.DMA] * NBUF,
    ],
)
def _cumsum_sc(x_hbm, out_hbm, inbuf, outbuf, insems, outsems):
    wid = lax.axis_index("s") * 2 + lax.axis_index("c")
    b = wid // (NW // B)
    l0 = (wid % (NW // B)) * LC

    def in_copy(t, slot):
        src = x_hbm.at[b, pl.ds(t * ST, ST), pl.ds(l0, LC)]
        return pltpu.make_async_copy(src, inbuf.at[slot], insems[slot])

    def out_copy(t, slot):
        dst = out_hbm.at[b, pl.ds(t * ST, ST), pl.ds(l0, LC)]
        return pltpu.make_async_copy(outbuf.at[slot], dst, outsems[slot])

    for slot in range(NBUF):
        in_copy(slot, slot).start()

    def group_body(g, carries):
        for slot in range(NBUF):
            t = NBUF * g + slot
            in_copy(t, slot).wait()

            @pl.when(g >= 1)
            def _wait_prev_out(slot=slot, t=t):
                out_copy(t - NBUF, slot).wait()

            def row_body(r, cs, slot=slot):
                new = []
                for j in range(NCHUNK):
                    c = cs[j] + inbuf[slot, r, pl.ds(j * 16, 16)]
                    outbuf[slot, r, pl.ds(j * 16, 16)] = c
                    new.append(c)
                return tuple(new)

            carries = lax.fori_loop(0, ST, row_body, carries, unroll=4)
            out_copy(t, slot).start()

            @pl.when(g + 1 < NGROUPS)
            def _prefetch(slot=slot, t=t):
                in_copy(t + NBUF, slot).start()

        return carries

    lax.fori_loop(0, NGROUPS, group_body,
                  tuple(jnp.zeros((16,), jnp.float32) for _ in range(NCHUNK)))

    for slot in range(NBUF):
        out_copy(NTILES - NBUF + slot, slot).wait()


def kernel(masks):
    return _cumsum_sc(masks)
